# baseline (speedup 1.0000x reference)
import jax
import jax.numpy as jnp
from jax.experimental import pallas as pl
from jax.experimental.pallas import tpu as pltpu

_TM = 1024
_TN = 512
_NCH = 8


def _fused_body(idx_ref, x_ref, w_hbm_ref, bias_ref, a_ref, bb_ref, o_ref,
                wb_ref, cbuf_ref, csem):
    bi = pl.program_id(0)
    mi = pl.program_id(1)
    dout, din = wb_ref.shape
    cs = dout // _NCH

    @pl.when((bi == 0) & (mi == 0))
    def _():
        copies = [
            pltpu.make_async_copy(
                w_hbm_ref.at[i * cs:(i + 1) * cs, :],
                cbuf_ref.at[i % 2], csem.at[i % 2])
            for i in range(_NCH)
        ]
        copies[0].start()
        copies[1].start()
        for i in range(_NCH):
            copies[i].wait()
            wb_ref[i * cs:(i + 1) * cs, :] = cbuf_ref[i % 2].astype(jnp.bfloat16)
            if i + 2 < _NCH:
                copies[i + 2].start()

    x = x_ref[0].astype(jnp.bfloat16)
    a = a_ref[0].astype(jnp.bfloat16)
    inter = jax.lax.dot_general(
        x, a, (((1,), (1,)), ((), ())),
        preferred_element_type=jnp.float32)
    ib = inter.astype(jnp.bfloat16)
    bb = bb_ref[0].astype(jnp.bfloat16)
    for n in range(0, dout, _TN):
        acc = jax.lax.dot_general(
            x, wb_ref[n:n + _TN, :], (((1,), (1,)), ((), ())),
            preferred_element_type=jnp.float32)
        lora = jax.lax.dot_general(
            ib, bb[n:n + _TN, :], (((1,), (1,)), ((), ())),
            preferred_element_type=jnp.float32)
        o_ref[0, :, n:n + _TN] = acc + lora + bias_ref[:, n:n + _TN]


def kernel(x, adapter_indices, W, b, lora_a, lora_b):
    B, S, DIN = x.shape
    DOUT = W.shape[0]
    E, R, _ = lora_a.shape
    idx = adapter_indices.astype(jnp.int32)
    bias = b.reshape(1, DOUT)

    grid = (B, S // _TM)

    grid_spec = pltpu.PrefetchScalarGridSpec(
        num_scalar_prefetch=1,
        grid=grid,
        in_specs=[
            pl.BlockSpec((1, _TM, DIN), lambda bi, mi, idx_ref: (bi, mi, 0)),
            pl.BlockSpec(memory_space=pltpu.MemorySpace.HBM),
            pl.BlockSpec((1, DOUT), lambda bi, mi, idx_ref: (0, 0)),
            pl.BlockSpec((1, R, DIN), lambda bi, mi, idx_ref: (idx_ref[bi], 0, 0)),
            pl.BlockSpec((1, DOUT, R), lambda bi, mi, idx_ref: (idx_ref[bi], 0, 0)),
        ],
        out_specs=pl.BlockSpec((1, _TM, DOUT), lambda bi, mi, idx_ref: (bi, mi, 0)),
        scratch_shapes=[
            pltpu.VMEM((DOUT, DIN), jnp.bfloat16),
            pltpu.VMEM((2, DOUT // _NCH, DIN), jnp.float32),
            pltpu.SemaphoreType.DMA((2,)),
        ],
    )

    return pl.pallas_call(
        _fused_body,
        grid_spec=grid_spec,
        out_shape=jax.ShapeDtypeStruct((B, S, DOUT), jnp.float32),
    )(idx, x, W, bias, lora_a, lora_b)


# speedup vs baseline: 1.0234x; 1.0234x over previous
import jax
import jax.numpy as jnp
from jax.experimental import pallas as pl
from jax.experimental.pallas import tpu as pltpu

_TM = 512
_TN = 256


def _fused_body(idx_ref, x_ref, w_ref, bias_ref, a_ref, bb_ref, o_ref, wb_ref):
    bi = pl.program_id(0)
    mi = pl.program_id(1)

    @pl.when((bi == 0) & (mi == 0))
    def _():
        wb_ref[...] = w_ref[...].astype(jnp.bfloat16)

    x = x_ref[0].astype(jnp.bfloat16)
    a = a_ref[0].astype(jnp.bfloat16)
    inter = jax.lax.dot_general(
        x, a, (((1,), (1,)), ((), ())),
        preferred_element_type=jnp.float32)
    ib = inter.astype(jnp.bfloat16)
    bb = bb_ref[0].astype(jnp.bfloat16)
    dout = bb.shape[0]
    for n in range(0, dout, _TN):
        acc = jax.lax.dot_general(
            x, wb_ref[n:n + _TN, :], (((1,), (1,)), ((), ())),
            preferred_element_type=jnp.float32)
        lora = jax.lax.dot_general(
            ib, bb[n:n + _TN, :], (((1,), (1,)), ((), ())),
            preferred_element_type=jnp.float32)
        o_ref[0, :, n:n + _TN] = acc + lora + bias_ref[:, n:n + _TN]


def kernel(x, adapter_indices, W, b, lora_a, lora_b):
    B, S, DIN = x.shape
    DOUT = W.shape[0]
    E, R, _ = lora_a.shape
    idx = adapter_indices.astype(jnp.int32)
    bias = b.reshape(1, DOUT)

    grid = (B, S // _TM)

    grid_spec = pltpu.PrefetchScalarGridSpec(
        num_scalar_prefetch=1,
        grid=grid,
        in_specs=[
            pl.BlockSpec((1, _TM, DIN), lambda bi, mi, idx_ref: (bi, mi, 0)),
            pl.BlockSpec((DOUT, DIN), lambda bi, mi, idx_ref: (0, 0)),
            pl.BlockSpec((1, DOUT), lambda bi, mi, idx_ref: (0, 0)),
            pl.BlockSpec((1, R, DIN), lambda bi, mi, idx_ref: (idx_ref[bi], 0, 0)),
            pl.BlockSpec((1, DOUT, R), lambda bi, mi, idx_ref: (idx_ref[bi], 0, 0)),
        ],
        out_specs=pl.BlockSpec((1, _TM, DOUT), lambda bi, mi, idx_ref: (bi, mi, 0)),
        scratch_shapes=[pltpu.VMEM((DOUT, DIN), jnp.bfloat16)],
    )

    return pl.pallas_call(
        _fused_body,
        grid_spec=grid_spec,
        out_shape=jax.ShapeDtypeStruct((B, S, DOUT), jnp.float32),
    )(idx, x, W, bias, lora_a, lora_b)


# speedup vs baseline: 1.0415x; 1.0176x over previous
import jax
import jax.numpy as jnp
from jax.experimental import pallas as pl
from jax.experimental.pallas import tpu as pltpu

_TM = 512
_TN = 1024


def _fused_body(idx_ref, x_ref, w_ref, bias_ref, a_ref, bb_ref, o_ref, wb_ref):
    bi = pl.program_id(0)
    mi = pl.program_id(1)

    @pl.when((bi == 0) & (mi == 0))
    def _():
        wb_ref[...] = w_ref[...].astype(jnp.bfloat16)

    x = x_ref[0].astype(jnp.bfloat16)
    a = a_ref[0].astype(jnp.bfloat16)
    inter = jax.lax.dot_general(
        x, a, (((1,), (1,)), ((), ())),
        preferred_element_type=jnp.float32)
    ib = inter.astype(jnp.bfloat16)
    bb = bb_ref[0].astype(jnp.bfloat16)
    dout = bb.shape[0]
    for n in range(0, dout, _TN):
        acc = jax.lax.dot_general(
            x, wb_ref[n:n + _TN, :], (((1,), (1,)), ((), ())),
            preferred_element_type=jnp.float32)
        lora = jax.lax.dot_general(
            ib, bb[n:n + _TN, :], (((1,), (1,)), ((), ())),
            preferred_element_type=jnp.float32)
        o_ref[0, :, n:n + _TN] = acc + lora + bias_ref[:, n:n + _TN]


def kernel(x, adapter_indices, W, b, lora_a, lora_b):
    B, S, DIN = x.shape
    DOUT = W.shape[0]
    E, R, _ = lora_a.shape
    idx = adapter_indices.astype(jnp.int32)
    bias = b.reshape(1, DOUT)

    grid = (B, S // _TM)

    grid_spec = pltpu.PrefetchScalarGridSpec(
        num_scalar_prefetch=1,
        grid=grid,
        in_specs=[
            pl.BlockSpec((1, _TM, DIN), lambda bi, mi, idx_ref: (bi, mi, 0)),
            pl.BlockSpec((DOUT, DIN), lambda bi, mi, idx_ref: (0, 0)),
            pl.BlockSpec((1, DOUT), lambda bi, mi, idx_ref: (0, 0)),
            pl.BlockSpec((1, R, DIN), lambda bi, mi, idx_ref: (idx_ref[bi], 0, 0)),
            pl.BlockSpec((1, DOUT, R), lambda bi, mi, idx_ref: (idx_ref[bi], 0, 0)),
        ],
        out_specs=pl.BlockSpec((1, _TM, DOUT), lambda bi, mi, idx_ref: (bi, mi, 0)),
        scratch_shapes=[pltpu.VMEM((DOUT, DIN), jnp.bfloat16)],
    )

    return pl.pallas_call(
        _fused_body,
        grid_spec=grid_spec,
        out_shape=jax.ShapeDtypeStruct((B, S, DOUT), jnp.float32),
    )(idx, x, W, bias, lora_a, lora_b)
